# baseline (speedup 1.0000x reference)
import functools

import jax
import jax.numpy as jnp
import numpy as np
from jax.experimental import pallas as pl
from jax.experimental.pallas import tpu as pltpu


def _upsample_matrix() -> np.ndarray:
    W = np.zeros((96, 48), np.float32)
    for j in range(96):
        c = j / 2 - 0.25
        k0 = int(np.floor(c))
        w1 = c - k0
        taps = [(k0, 1.0 - w1), (k0 + 1, w1)]
        valid = [(k, w) for k, w in taps if 0 <= k < 48]
        s = sum(w for _, w in valid)
        for k, w in valid:
            W[j, k] = w / s
    return W


_W96x48 = _upsample_matrix()

_COLS = 48 * 48
_BH1 = 24


def _mean_body(vol_ref, mean_ref):
    s = jnp.sum(vol_ref[0], axis=(-2, -1)) * (1.0 / _COLS)
    mean_ref[...] = s[None]


def _head_body(m_ref, w_ref, map_ref, vol_ref, loss_ref):
    w = w_ref[...]
    acc = jnp.float32(0.0)
    for b in range(8):
        t = jax.lax.dot(w, m_ref[b], precision=jax.lax.Precision.HIGHEST)
        up = jax.lax.dot_general(
            t, w, (((1,), (1,)), ((), ())),
            precision=jax.lax.Precision.HIGHEST)
        vol_ref[b] = up
        d = up - map_ref[b]
        acc = acc + jnp.sum(d * d)
    loss_ref[0, 0] = acc * (1.0 / (8 * 96 * 96))


@jax.jit
def kernel(out_volume, out_map, label):
    del label

    m = pl.pallas_call(
        _mean_body,
        grid=(8, 48 // _BH1),
        in_specs=[pl.BlockSpec((1, _BH1, 48, 48, 48),
                               lambda b, i: (b, i, 0, 0, 0))],
        out_specs=pl.BlockSpec((1, _BH1, 48), lambda b, i: (b, i, 0)),
        out_shape=jax.ShapeDtypeStruct((8, 48, 48), jnp.float32),
    )(out_volume)

    wmat = jnp.asarray(_W96x48)
    map3 = out_map.reshape(8, 96, 96)

    out_vol, loss = pl.pallas_call(
        _head_body,
        in_specs=[
            pl.BlockSpec((8, 48, 48), lambda: (0, 0, 0)),
            pl.BlockSpec((96, 48), lambda: (0, 0)),
            pl.BlockSpec((8, 96, 96), lambda: (0, 0, 0)),
        ],
        out_specs=[
            pl.BlockSpec((8, 96, 96), lambda: (0, 0, 0)),
            pl.BlockSpec(memory_space=pltpu.SMEM),
        ],
        out_shape=[
            jax.ShapeDtypeStruct((8, 96, 96), jnp.float32),
            jax.ShapeDtypeStruct((1, 1), jnp.float32),
        ],
    )(m, wmat, map3)

    return loss[0, 0], out_vol


# speedup vs baseline: 1.0198x; 1.0198x over previous
import functools

import jax
import jax.numpy as jnp
import numpy as np
from jax.experimental import pallas as pl
from jax.experimental.pallas import tpu as pltpu


def _upsample_matrix() -> np.ndarray:
    W = np.zeros((96, 48), np.float32)
    for j in range(96):
        c = j / 2 - 0.25
        k0 = int(np.floor(c))
        w1 = c - k0
        taps = [(k0, 1.0 - w1), (k0 + 1, w1)]
        valid = [(k, w) for k, w in taps if 0 <= k < 48]
        s = sum(w for _, w in valid)
        for k, w in valid:
            W[j, k] = w / s
    return W


_W96x48 = _upsample_matrix()

_COLS = 48 * 48
_BH1 = 8


def _mean_body(vol_ref, mean_ref):
    s = jnp.sum(vol_ref[0], axis=(-2, -1)) * (1.0 / _COLS)
    mean_ref[...] = s[None]


def _head_body(m_ref, w_ref, map_ref, vol_ref, loss_ref):
    w = w_ref[...]
    acc = jnp.float32(0.0)
    for b in range(8):
        t = jax.lax.dot(w, m_ref[b], precision=jax.lax.Precision.HIGHEST)
        up = jax.lax.dot_general(
            t, w, (((1,), (1,)), ((), ())),
            precision=jax.lax.Precision.HIGHEST)
        vol_ref[b] = up
        d = up - map_ref[b]
        acc = acc + jnp.sum(d * d)
    loss_ref[0, 0] = acc * (1.0 / (8 * 96 * 96))


@jax.jit
def kernel(out_volume, out_map, label):
    del label

    m = pl.pallas_call(
        _mean_body,
        grid=(8, 48 // _BH1),
        in_specs=[pl.BlockSpec((1, _BH1, 48, 48, 48),
                               lambda b, i: (b, i, 0, 0, 0))],
        out_specs=pl.BlockSpec((1, _BH1, 48), lambda b, i: (b, i, 0)),
        out_shape=jax.ShapeDtypeStruct((8, 48, 48), jnp.float32),
    )(out_volume)

    wmat = jnp.asarray(_W96x48)
    map3 = out_map.reshape(8, 96, 96)

    out_vol, loss = pl.pallas_call(
        _head_body,
        in_specs=[
            pl.BlockSpec((8, 48, 48), lambda: (0, 0, 0)),
            pl.BlockSpec((96, 48), lambda: (0, 0)),
            pl.BlockSpec((8, 96, 96), lambda: (0, 0, 0)),
        ],
        out_specs=[
            pl.BlockSpec((8, 96, 96), lambda: (0, 0, 0)),
            pl.BlockSpec(memory_space=pltpu.SMEM),
        ],
        out_shape=[
            jax.ShapeDtypeStruct((8, 96, 96), jnp.float32),
            jax.ShapeDtypeStruct((1, 1), jnp.float32),
        ],
    )(m, wmat, map3)

    return loss[0, 0], out_vol
